# Initial kernel scaffold; baseline (speedup 1.0000x reference)
import jax
import jax.numpy as jnp
from jax.experimental import pallas as pl


def kernel(student_features, teacher_features, teacher_codes, codebook):
    raise NotImplementedError("write your pallas kernel here")


# speedup vs baseline: 5.8215x; 5.8215x over previous
import functools

import jax
import jax.numpy as jnp
from jax import lax
from jax.experimental import pallas as pl

_B, _T, _D, _K = 4, 1024, 32, 8192
_N = _B * _T
_NUM_NEG = 16
_TEMP = 0.1
_MARGIN = 0.2
_FEAT_W = 1.0
_TRIP_W = 1.0
_CONTR_W = 0.5
_BN = 128
_NB = _N // _BN
_EPS = 1e-12


def _block_body(s_ref, t_ref, nt_ref, code_ref, cb_ref, out_ref):
    s = s_ref[...]
    t = t_ref[...]
    nt = nt_ref[...]
    code = code_ref[...]
    cb = cb_ref[...]

    diff = s - t
    sq = jnp.sum(diff * diff, axis=1, keepdims=True)
    feat = sq / _D
    pos_dist = jnp.sqrt(sq)
    dn = s - nt
    neg_dist = jnp.sqrt(jnp.sum(dn * dn, axis=1, keepdims=True))
    trip = jnp.maximum(pos_dist - neg_dist + _MARGIN, 0.0)

    S = lax.dot_general(s, cb, (((1,), (1,)), ((), ())),
                        preferred_element_type=jnp.float32)
    x2 = jnp.sum(s * s, axis=1, keepdims=True)
    cb2 = cb * cb
    ones_row = jnp.ones((1, _D), dtype=jnp.float32)
    c2 = lax.dot_general(ones_row, cb2, (((1,), (1,)), ((), ())),
                         preferred_element_type=jnp.float32)
    rx = 1.0 / jnp.maximum(jnp.sqrt(x2), _EPS)
    rc = 1.0 / jnp.maximum(jnp.sqrt(c2), _EPS)
    cosmat = S * rx * rc
    d2 = (x2 + c2) - 2.0 * S

    col = lax.broadcasted_iota(jnp.int32, (_BN, _K), 1)
    is_pos = col == code
    d2 = jnp.where(is_pos, jnp.inf, d2)
    poscos = jnp.sum(jnp.where(is_pos, cosmat, 0.0), axis=1, keepdims=True)

    negs = []
    for _ in range(_NUM_NEG):
        m = jnp.min(d2, axis=1, keepdims=True)
        idx = jnp.min(jnp.where(d2 == m, col, _K), axis=1, keepdims=True)
        onehot = col == idx
        negs.append(jnp.sum(jnp.where(onehot, cosmat, 0.0),
                            axis=1, keepdims=True))
        d2 = jnp.where(onehot, jnp.inf, d2)

    logits = [poscos / _TEMP] + [n / _TEMP for n in negs]
    mmax = functools.reduce(jnp.maximum, logits)
    sumexp = sum(jnp.exp(l - mmax) for l in logits)
    ce = mmax + jnp.log(sumexp) - logits[0]

    feat_part = jnp.sum(feat)
    trip_part = jnp.sum(trip)
    ce_part = jnp.sum(ce)

    r = lax.broadcasted_iota(jnp.int32, (1, 8, 128), 1)
    c = lax.broadcasted_iota(jnp.int32, (1, 8, 128), 2)
    partial = (jnp.where((r == 0) & (c == 0), feat_part, 0.0)
               + jnp.where((r == 0) & (c == 1), trip_part, 0.0)
               + jnp.where((r == 0) & (c == 2), ce_part, 0.0))
    out_ref[...] = partial


def _fused_loss(student_flat, teacher_flat, negteacher_flat, codes2d, codebook):
    out = pl.pallas_call(
        _block_body,
        grid=(_NB,),
        in_specs=[
            pl.BlockSpec((_BN, _D), lambda i: (i, 0)),
            pl.BlockSpec((_BN, _D), lambda i: (i, 0)),
            pl.BlockSpec((_BN, _D), lambda i: (i, 0)),
            pl.BlockSpec((_BN, 1), lambda i: (i, 0)),
            pl.BlockSpec((_K, _D), lambda i: (0, 0)),
        ],
        out_specs=pl.BlockSpec((1, 8, 128), lambda i: (i, 0, 0)),
        out_shape=jax.ShapeDtypeStruct((_NB, 8, 128), jnp.float32),
    )(student_flat, teacher_flat, negteacher_flat, codes2d, codebook)
    return out


def kernel(student_features, teacher_features, teacher_codes, codebook):
    student_flat = student_features.reshape(_N, _D)
    teacher_flat = teacher_features.reshape(_N, _D)
    negteacher_flat = jnp.roll(teacher_features, shift=1, axis=0).reshape(_N, _D)
    codes2d = teacher_codes.reshape(_N, 1).astype(jnp.int32)

    parts = _fused_loss(student_flat, teacher_flat, negteacher_flat,
                        codes2d, codebook)
    sums = jnp.sum(parts, axis=0)
    feat_sum = sums[0, 0]
    trip_sum = sums[0, 1]
    ce_sum = sums[0, 2]
    inv_n = 1.0 / _N
    return (_FEAT_W * feat_sum * inv_n
            + _TRIP_W * trip_sum * inv_n
            + _CONTR_W * ce_sum * inv_n)


# speedup vs baseline: 7.7107x; 1.3245x over previous
import functools

import jax
import jax.numpy as jnp
from jax import lax
from jax.experimental import pallas as pl

_B, _T, _D, _K = 4, 1024, 32, 8192
_N = _B * _T
_NUM_NEG = 16
_TEMP = 0.1
_MARGIN = 0.2
_FEAT_W = 1.0
_TRIP_W = 1.0
_CONTR_W = 0.5
_BN = 128
_NB = _N // _BN
_EPS = 1e-12


def _block_body(s_ref, t_ref, nt_ref, code_ref, cb_ref, out_ref):
    s = s_ref[...]
    t = t_ref[...]
    nt = nt_ref[...]
    code = code_ref[...]
    cb = cb_ref[...]

    diff = s - t
    sq = jnp.sum(diff * diff, axis=1, keepdims=True)
    feat = sq / _D
    pos_dist = jnp.sqrt(sq)
    dn = s - nt
    neg_dist = jnp.sqrt(jnp.sum(dn * dn, axis=1, keepdims=True))
    trip = jnp.maximum(pos_dist - neg_dist + _MARGIN, 0.0)

    S = lax.dot_general(s, cb, (((1,), (1,)), ((), ())),
                        preferred_element_type=jnp.float32)
    x2 = jnp.sum(s * s, axis=1, keepdims=True)
    cb2 = cb * cb
    ones_row = jnp.ones((1, _D), dtype=jnp.float32)
    c2 = lax.dot_general(ones_row, cb2, (((1,), (1,)), ((), ())),
                         preferred_element_type=jnp.float32)
    rx = 1.0 / jnp.maximum(jnp.sqrt(x2), _EPS)
    rc = 1.0 / jnp.maximum(jnp.sqrt(c2), _EPS)
    cosmat = S * rx * rc
    d2 = (x2 + c2) - 2.0 * S

    col = lax.broadcasted_iota(jnp.int32, (_BN, _K), 1)
    is_pos = col == code
    d2 = jnp.where(is_pos, jnp.inf, d2)
    poscos = jnp.sum(jnp.where(is_pos, cosmat, 0.0), axis=1, keepdims=True)

    d2 = jnp.maximum(d2, 0.0)
    kq = lax.bitcast_convert_type(d2, jnp.int32)
    kq = jnp.bitwise_or(jnp.bitwise_and(kq, jnp.int32(~(_K - 1))), col)
    imax = jnp.int32(jnp.iinfo(jnp.int32).max)
    gm = jnp.min(kq, axis=1, keepdims=True)
    negs = []
    for i in range(_NUM_NEG):
        hit = kq == gm
        negs.append(jnp.sum(jnp.where(hit, cosmat, 0.0),
                            axis=1, keepdims=True))
        if i < _NUM_NEG - 1:
            gm = jnp.min(jnp.where(kq > gm, kq, imax),
                         axis=1, keepdims=True)

    logits = [poscos / _TEMP] + [n / _TEMP for n in negs]
    mmax = functools.reduce(jnp.maximum, logits)
    sumexp = sum(jnp.exp(l - mmax) for l in logits)
    ce = mmax + jnp.log(sumexp) - logits[0]

    feat_part = jnp.sum(feat)
    trip_part = jnp.sum(trip)
    ce_part = jnp.sum(ce)

    r = lax.broadcasted_iota(jnp.int32, (1, 8, 128), 1)
    c = lax.broadcasted_iota(jnp.int32, (1, 8, 128), 2)
    partial = (jnp.where((r == 0) & (c == 0), feat_part, 0.0)
               + jnp.where((r == 0) & (c == 1), trip_part, 0.0)
               + jnp.where((r == 0) & (c == 2), ce_part, 0.0))
    out_ref[...] = partial


def _fused_loss(student_flat, teacher_flat, negteacher_flat, codes2d, codebook):
    out = pl.pallas_call(
        _block_body,
        grid=(_NB,),
        in_specs=[
            pl.BlockSpec((_BN, _D), lambda i: (i, 0)),
            pl.BlockSpec((_BN, _D), lambda i: (i, 0)),
            pl.BlockSpec((_BN, _D), lambda i: (i, 0)),
            pl.BlockSpec((_BN, 1), lambda i: (i, 0)),
            pl.BlockSpec((_K, _D), lambda i: (0, 0)),
        ],
        out_specs=pl.BlockSpec((1, 8, 128), lambda i: (i, 0, 0)),
        out_shape=jax.ShapeDtypeStruct((_NB, 8, 128), jnp.float32),
    )(student_flat, teacher_flat, negteacher_flat, codes2d, codebook)
    return out


def kernel(student_features, teacher_features, teacher_codes, codebook):
    student_flat = student_features.reshape(_N, _D)
    teacher_flat = teacher_features.reshape(_N, _D)
    negteacher_flat = jnp.roll(teacher_features, shift=1, axis=0).reshape(_N, _D)
    codes2d = teacher_codes.reshape(_N, 1).astype(jnp.int32)

    parts = _fused_loss(student_flat, teacher_flat, negteacher_flat,
                        codes2d, codebook)
    sums = jnp.sum(parts, axis=0)
    feat_sum = sums[0, 0]
    trip_sum = sums[0, 1]
    ce_sum = sums[0, 2]
    inv_n = 1.0 / _N
    return (_FEAT_W * feat_sum * inv_n
            + _TRIP_W * trip_sum * inv_n
            + _CONTR_W * ce_sum * inv_n)


# speedup vs baseline: 10.1330x; 1.3141x over previous
import functools

import jax
import jax.numpy as jnp
from jax import lax
from jax.experimental import pallas as pl

_B, _T, _D, _K = 4, 1024, 32, 8192
_N = _B * _T
_NUM_NEG = 16
_TEMP = 0.1
_MARGIN = 0.2
_FEAT_W = 1.0
_TRIP_W = 1.0
_CONTR_W = 0.5
_BN = 128
_NB = _N // _BN
_EPS = 1e-12
_QBITS = 13
_QMASK = (1 << _QBITS) - 1
_QSCALE = 4095.0


def _block_body(s_ref, t_ref, nt_ref, code_ref, cb_ref, out_ref):
    s = s_ref[...]
    t = t_ref[...]
    nt = nt_ref[...]
    code = code_ref[...]
    cb = cb_ref[...]

    diff = s - t
    sq = jnp.sum(diff * diff, axis=1, keepdims=True)
    feat = sq / _D
    pos_dist = jnp.sqrt(sq)
    dn = s - nt
    neg_dist = jnp.sqrt(jnp.sum(dn * dn, axis=1, keepdims=True))
    trip = jnp.maximum(pos_dist - neg_dist + _MARGIN, 0.0)

    c2col = jnp.sum(cb * cb, axis=1, keepdims=True)
    rc_col = 1.0 / jnp.maximum(jnp.sqrt(c2col), _EPS)
    x2 = jnp.sum(s * s, axis=1, keepdims=True)
    rx = 1.0 / jnp.maximum(jnp.sqrt(x2), _EPS)

    s_aug = jnp.concatenate([-2.0 * s, jnp.ones((_BN, 1), jnp.float32)],
                            axis=1)
    cb_aug = jnp.concatenate([cb, c2col], axis=1)
    c2m2s = lax.dot_general(s_aug, cb_aug, (((1,), (1,)), ((), ())),
                            preferred_element_type=jnp.float32)
    sn = s * rx
    cbn = cb * rc_col
    cos = lax.dot_general(sn, cbn, (((1,), (1,)), ((), ())),
                          preferred_element_type=jnp.float32)

    col = lax.broadcasted_iota(jnp.int32, (_BN, _K), 1)
    is_pos = col == code
    poscos = jnp.sum(jnp.where(is_pos, cos, 0.0), axis=1, keepdims=True)

    d2 = jnp.maximum(c2m2s + x2, 0.0)
    d2 = jnp.where(is_pos, jnp.inf, d2)
    cosq = jnp.maximum(cos * _QSCALE + _QSCALE, 0.0).astype(jnp.int32)
    kq = jnp.bitwise_or(
        jnp.bitwise_and(lax.bitcast_convert_type(d2, jnp.int32),
                        jnp.int32(~_QMASK)),
        cosq)

    imax = jnp.int32(jnp.iinfo(jnp.int32).max)
    gms = []
    gm = jnp.min(kq, axis=1, keepdims=True)
    gms.append(gm)
    for _ in range(_NUM_NEG - 1):
        gm = jnp.min(jnp.where(kq > gm, kq, imax), axis=1, keepdims=True)
        gms.append(gm)

    inv_q = jnp.float32(1.0 / _QSCALE)
    logits = [poscos / _TEMP]
    for gm_i in gms:
        cq = jnp.bitwise_and(gm_i, jnp.int32(_QMASK)).astype(jnp.float32)
        logits.append((cq * inv_q - 1.0) / _TEMP)
    mmax = functools.reduce(jnp.maximum, logits)
    sumexp = sum(jnp.exp(l - mmax) for l in logits)
    ce = mmax + jnp.log(sumexp) - logits[0]

    feat_part = jnp.sum(feat)
    trip_part = jnp.sum(trip)
    ce_part = jnp.sum(ce)

    r = lax.broadcasted_iota(jnp.int32, (1, 8, 128), 1)
    c = lax.broadcasted_iota(jnp.int32, (1, 8, 128), 2)
    partial = (jnp.where((r == 0) & (c == 0), feat_part, 0.0)
               + jnp.where((r == 0) & (c == 1), trip_part, 0.0)
               + jnp.where((r == 0) & (c == 2), ce_part, 0.0))
    out_ref[...] = partial


def _fused_loss(student_flat, teacher_flat, negteacher_flat, codes2d, codebook):
    out = pl.pallas_call(
        _block_body,
        grid=(_NB,),
        in_specs=[
            pl.BlockSpec((_BN, _D), lambda i: (i, 0)),
            pl.BlockSpec((_BN, _D), lambda i: (i, 0)),
            pl.BlockSpec((_BN, _D), lambda i: (i, 0)),
            pl.BlockSpec((_BN, 1), lambda i: (i, 0)),
            pl.BlockSpec((_K, _D), lambda i: (0, 0)),
        ],
        out_specs=pl.BlockSpec((1, 8, 128), lambda i: (i, 0, 0)),
        out_shape=jax.ShapeDtypeStruct((_NB, 8, 128), jnp.float32),
    )(student_flat, teacher_flat, negteacher_flat, codes2d, codebook)
    return out


def kernel(student_features, teacher_features, teacher_codes, codebook):
    student_flat = student_features.reshape(_N, _D)
    teacher_flat = teacher_features.reshape(_N, _D)
    negteacher_flat = jnp.roll(teacher_features, shift=1, axis=0).reshape(_N, _D)
    codes2d = teacher_codes.reshape(_N, 1).astype(jnp.int32)

    parts = _fused_loss(student_flat, teacher_flat, negteacher_flat,
                        codes2d, codebook)
    sums = jnp.sum(parts, axis=0)
    feat_sum = sums[0, 0]
    trip_sum = sums[0, 1]
    ce_sum = sums[0, 2]
    inv_n = 1.0 / _N
    return (_FEAT_W * feat_sum * inv_n
            + _TRIP_W * trip_sum * inv_n
            + _CONTR_W * ce_sum * inv_n)


# speedup vs baseline: 11.6357x; 1.1483x over previous
import functools

import jax
import jax.numpy as jnp
from jax import lax
from jax.experimental import pallas as pl

_B, _T, _D, _K = 4, 1024, 32, 8192
_N = _B * _T
_NUM_NEG = 16
_TEMP = 0.1
_MARGIN = 0.2
_FEAT_W = 1.0
_TRIP_W = 1.0
_CONTR_W = 0.5
_BN = 128
_NB = _N // _BN
_EPS = 1e-12
_QBITS = 13
_QMASK = (1 << _QBITS) - 1
_QSCALE = 4095.0


def _block_body(s_ref, t_ref, nt_ref, code_ref, cb_ref, out_ref):
    s = s_ref[...]
    t = t_ref[...]
    nt = nt_ref[...]
    code = code_ref[...]
    cb = cb_ref[...]

    diff = s - t
    sq = jnp.sum(diff * diff, axis=1, keepdims=True)
    feat = sq / _D
    pos_dist = jnp.sqrt(sq)
    dn = s - nt
    neg_dist = jnp.sqrt(jnp.sum(dn * dn, axis=1, keepdims=True))
    trip = jnp.maximum(pos_dist - neg_dist + _MARGIN, 0.0)

    c2col = jnp.sum(cb * cb, axis=1, keepdims=True)
    rc_col = 1.0 / jnp.maximum(jnp.sqrt(c2col), _EPS)
    x2 = jnp.sum(s * s, axis=1, keepdims=True)
    rx = 1.0 / jnp.maximum(jnp.sqrt(x2), _EPS)

    s_aug = jnp.concatenate([-2.0 * s, jnp.ones((_BN, 1), jnp.float32)],
                            axis=1)
    cb_aug = jnp.concatenate([cb, c2col], axis=1)
    c2m2s = lax.dot_general(s_aug, cb_aug, (((1,), (1,)), ((), ())),
                            preferred_element_type=jnp.float32)
    sn = s * rx
    cbn = cb * rc_col
    cos = lax.dot_general(sn, cbn, (((1,), (1,)), ((), ())),
                          preferred_element_type=jnp.float32)

    col = lax.broadcasted_iota(jnp.int32, (_BN, _K), 1)
    is_pos = col == code
    poscos = jnp.sum(jnp.where(is_pos, cos, 0.0), axis=1, keepdims=True)

    d2 = jnp.maximum(c2m2s + x2, 0.0)
    d2 = jnp.where(is_pos, jnp.inf, d2)
    cosq = jnp.maximum(cos * _QSCALE + _QSCALE, 0.0).astype(jnp.int32)
    kq = jnp.bitwise_or(
        jnp.bitwise_and(lax.bitcast_convert_type(d2, jnp.int32),
                        jnp.int32(~_QMASK)),
        cosq)

    imax = jnp.int32(jnp.iinfo(jnp.int32).max)
    gms = []
    gm = jnp.min(kq, axis=1, keepdims=True)
    gms.append(gm)
    for _ in range(_NUM_NEG - 1):
        shift = imax - gm
        v = kq + shift
        vmin = jnp.min(v, axis=1, keepdims=True)
        gm = vmin - shift
        gms.append(gm)

    inv_q = jnp.float32(1.0 / _QSCALE)
    logits = [poscos / _TEMP]
    for gm_i in gms:
        cq = jnp.bitwise_and(gm_i, jnp.int32(_QMASK)).astype(jnp.float32)
        logits.append((cq * inv_q - 1.0) / _TEMP)
    mmax = functools.reduce(jnp.maximum, logits)
    sumexp = sum(jnp.exp(l - mmax) for l in logits)
    ce = mmax + jnp.log(sumexp) - logits[0]

    feat_part = jnp.sum(feat)
    trip_part = jnp.sum(trip)
    ce_part = jnp.sum(ce)

    r = lax.broadcasted_iota(jnp.int32, (1, 8, 128), 1)
    c = lax.broadcasted_iota(jnp.int32, (1, 8, 128), 2)
    partial = (jnp.where((r == 0) & (c == 0), feat_part, 0.0)
               + jnp.where((r == 0) & (c == 1), trip_part, 0.0)
               + jnp.where((r == 0) & (c == 2), ce_part, 0.0))
    out_ref[...] = partial


def _fused_loss(student_flat, teacher_flat, negteacher_flat, codes2d, codebook):
    out = pl.pallas_call(
        _block_body,
        grid=(_NB,),
        in_specs=[
            pl.BlockSpec((_BN, _D), lambda i: (i, 0)),
            pl.BlockSpec((_BN, _D), lambda i: (i, 0)),
            pl.BlockSpec((_BN, _D), lambda i: (i, 0)),
            pl.BlockSpec((_BN, 1), lambda i: (i, 0)),
            pl.BlockSpec((_K, _D), lambda i: (0, 0)),
        ],
        out_specs=pl.BlockSpec((1, 8, 128), lambda i: (i, 0, 0)),
        out_shape=jax.ShapeDtypeStruct((_NB, 8, 128), jnp.float32),
    )(student_flat, teacher_flat, negteacher_flat, codes2d, codebook)
    return out


def kernel(student_features, teacher_features, teacher_codes, codebook):
    student_flat = student_features.reshape(_N, _D)
    teacher_flat = teacher_features.reshape(_N, _D)
    negteacher_flat = jnp.roll(teacher_features, shift=1, axis=0).reshape(_N, _D)
    codes2d = teacher_codes.reshape(_N, 1).astype(jnp.int32)

    parts = _fused_loss(student_flat, teacher_flat, negteacher_flat,
                        codes2d, codebook)
    sums = jnp.sum(parts, axis=0)
    feat_sum = sums[0, 0]
    trip_sum = sums[0, 1]
    ce_sum = sums[0, 2]
    inv_n = 1.0 / _N
    return (_FEAT_W * feat_sum * inv_n
            + _TRIP_W * trip_sum * inv_n
            + _CONTR_W * ce_sum * inv_n)


# speedup vs baseline: 12.3633x; 1.0625x over previous
import functools

import jax
import jax.numpy as jnp
from jax import lax
from jax.experimental import pallas as pl

_B, _T, _D, _K = 4, 1024, 32, 8192
_N = _B * _T
_NUM_NEG = 16
_TEMP = 0.1
_MARGIN = 0.2
_FEAT_W = 1.0
_TRIP_W = 1.0
_CONTR_W = 0.5
_BN = 128
_NB = _N // _BN
_EPS = 1e-12
_QBITS = 13
_QMASK = (1 << _QBITS) - 1
_QSCALE = 4095.0
_NSPLIT = 4
_KQ = _K // _NSPLIT


def _extract16(kq, imax):
    gms = []
    gm = jnp.min(kq, axis=0, keepdims=True)
    gms.append(gm)
    for _ in range(_NUM_NEG - 1):
        shift = imax - gm
        v = kq + shift
        gm = jnp.min(v, axis=0, keepdims=True) - shift
        gms.append(gm)
    return gms


def _block_body(s_ref, t_ref, nt_ref, code_ref, cb_ref, out_ref):
    s = s_ref[...]
    t = t_ref[...]
    nt = nt_ref[...]
    code = code_ref[...].reshape(1, _BN)
    cb = cb_ref[...]

    diff = s - t
    sq = jnp.sum(diff * diff, axis=1, keepdims=True)
    feat = sq / _D
    pos_dist = jnp.sqrt(sq)
    dn = s - nt
    neg_dist = jnp.sqrt(jnp.sum(dn * dn, axis=1, keepdims=True))
    trip = jnp.maximum(pos_dist - neg_dist + _MARGIN, 0.0)

    c2col = jnp.sum(cb * cb, axis=1, keepdims=True)
    rc_col = 1.0 / jnp.maximum(jnp.sqrt(c2col), _EPS)
    x2 = jnp.sum(s * s, axis=1, keepdims=True)
    rx = 1.0 / jnp.maximum(jnp.sqrt(x2), _EPS)

    ones_bn = jnp.ones((_BN, 1), jnp.float32)
    s_aug = jnp.concatenate([-2.0 * s, ones_bn, x2], axis=1)
    ones_k = jnp.ones((_K, 1), jnp.float32)
    cb_aug = jnp.concatenate([cb, c2col, ones_k], axis=1)
    d2 = lax.dot_general(cb_aug, s_aug, (((1,), (1,)), ((), ())),
                         preferred_element_type=jnp.float32)
    sn = s * rx
    cbn = cb * rc_col
    cos = lax.dot_general(cbn, sn, (((1,), (1,)), ((), ())),
                          preferred_element_type=jnp.float32)

    row = lax.broadcasted_iota(jnp.int32, (_K, _BN), 0)
    is_pos = row == code
    d2 = jnp.maximum(d2, 0.0)
    d2 = jnp.where(is_pos, jnp.inf, d2)
    cosq = jnp.maximum(cos * _QSCALE + _QSCALE, 0.0).astype(jnp.int32)
    kq = jnp.bitwise_or(
        jnp.bitwise_and(lax.bitcast_convert_type(d2, jnp.int32),
                        jnp.int32(~_QMASK)),
        cosq)

    posk = jnp.max(kq, axis=0, keepdims=True)

    imax = jnp.int32(jnp.iinfo(jnp.int32).max)
    cand = []
    for q in range(_NSPLIT):
        cand += _extract16(kq[q * _KQ:(q + 1) * _KQ, :], imax)
    cand64 = jnp.concatenate(cand, axis=0)
    gms = _extract16(cand64, imax)

    inv_q = jnp.float32(1.0 / (_QSCALE * _TEMP))
    off = jnp.float32(-1.0 / _TEMP)
    qmask = jnp.int32(_QMASK)
    logits = [jnp.bitwise_and(posk, qmask).astype(jnp.float32) * inv_q + off]
    for gm_i in gms:
        logits.append(
            jnp.bitwise_and(gm_i, qmask).astype(jnp.float32) * inv_q + off)
    mmax = functools.reduce(jnp.maximum, logits)
    sumexp = sum(jnp.exp(l - mmax) for l in logits)
    ce = mmax + jnp.log(sumexp) - logits[0]

    feat_part = jnp.sum(feat)
    trip_part = jnp.sum(trip)
    ce_part = jnp.sum(ce)

    r = lax.broadcasted_iota(jnp.int32, (1, 8, 128), 1)
    c = lax.broadcasted_iota(jnp.int32, (1, 8, 128), 2)
    partial = (jnp.where((r == 0) & (c == 0), feat_part, 0.0)
               + jnp.where((r == 0) & (c == 1), trip_part, 0.0)
               + jnp.where((r == 0) & (c == 2), ce_part, 0.0))
    out_ref[...] = partial


def _fused_loss(student_flat, teacher_flat, negteacher_flat, codes3d, codebook):
    out = pl.pallas_call(
        _block_body,
        grid=(_NB,),
        in_specs=[
            pl.BlockSpec((_BN, _D), lambda i: (i, 0)),
            pl.BlockSpec((_BN, _D), lambda i: (i, 0)),
            pl.BlockSpec((_BN, _D), lambda i: (i, 0)),
            pl.BlockSpec((1, 1, _BN), lambda i: (i, 0, 0)),
            pl.BlockSpec((_K, _D), lambda i: (0, 0)),
        ],
        out_specs=pl.BlockSpec((1, 8, 128), lambda i: (i, 0, 0)),
        out_shape=jax.ShapeDtypeStruct((_NB, 8, 128), jnp.float32),
    )(student_flat, teacher_flat, negteacher_flat, codes3d, codebook)
    return out


def kernel(student_features, teacher_features, teacher_codes, codebook):
    student_flat = student_features.reshape(_N, _D)
    teacher_flat = teacher_features.reshape(_N, _D)
    negteacher_flat = jnp.roll(teacher_features, shift=1, axis=0).reshape(_N, _D)
    codes3d = teacher_codes.reshape(_NB, 1, _BN).astype(jnp.int32)

    parts = _fused_loss(student_flat, teacher_flat, negteacher_flat,
                        codes3d, codebook)
    sums = jnp.sum(parts, axis=0)
    feat_sum = sums[0, 0]
    trip_sum = sums[0, 1]
    ce_sum = sums[0, 2]
    inv_n = 1.0 / _N
    return (_FEAT_W * feat_sum * inv_n
            + _TRIP_W * trip_sum * inv_n
            + _CONTR_W * ce_sum * inv_n)
